# baseline (speedup 1.0000x reference)
import jax
import jax.numpy as jnp
from jax import lax
from jax.experimental import pallas as pl
from jax.experimental.pallas import tpu as pltpu
from jax.experimental.pallas import tpu_sc as plsc

N_NODE = 10000
NE = 320000
D = 128
D_OUT = 64
NS = 16
K = 40
ROWS_PER_TILE = 624
TAIL_ROWS = N_NODE - NS * ROWS_PER_TILE
TAIL_BASE = NS * ROWS_PER_TILE
BC = 16


def _stripe(fn, s):
    fn(s * ROWS_PER_TILE, ROWS_PER_TILE)

    @pl.when(s == NS - 1)
    def _():
        fn(TAIL_BASE, TAIL_ROWS)


def _zero_acc(zb, acc_sh, s):
    zv = jnp.zeros((16,), jnp.float32)

    def zrow(i, c):
        for j in range(D // 16):
            zb[i, pl.ds(j * 16, 16)] = zv
        return c
    lax.fori_loop(0, BC, zrow, 0)

    def z(r0, n):
        if n == ROWS_PER_TILE:
            for j in range(ROWS_PER_TILE // BC):
                pltpu.sync_copy(zb, acc_sh.at[pl.ds(r0 + j * BC, BC)])
        else:
            pltpu.sync_copy(zb.at[pl.ds(0, n)], acc_sh.at[pl.ds(r0, n)])
    _stripe(z, s)


S = 5
SK = K * S


def _idx_issue(esrc_ref, edst_ref, ibs, ibd, isem, pset, off):
    pltpu.async_copy(esrc_ref.at[pl.ds(off, SK)],
                     ibs.at[pl.ds(pset * SK, SK)], isem.at[pset])
    pltpu.async_copy(edst_ref.at[pl.ds(off, SK)],
                     ibd.at[pl.ds(pset * SK, SK)], isem.at[pset])


def _idx_wait(esrc_ref, edst_ref, ibs, ibd, isem, pset, off):
    pltpu.make_async_copy(esrc_ref.at[pl.ds(off, SK)],
                          ibs.at[pl.ds(pset * SK, SK)], isem.at[pset]).wait()
    pltpu.make_async_copy(edst_ref.at[pl.ds(off, SK)],
                          ibd.at[pl.ds(pset * SK, SK)], isem.at[pset]).wait()


def _hist_update(hist, ibd, pset, one16):
    for b in range(S):
        base = pset * SK + b * K
        for j in range(K // 16):
            iv = ibd[pl.ds(base + j * 16, 16)]
            plsc.addupdate_scatter(hist, [iv], one16)
        if K % 16:
            iv = ibd[pl.ds(base + K - 16, 16)]
            tail_mask = lax.iota(jnp.int32, 16) >= (16 - K % 16)
            plsc.addupdate_scatter(hist, [iv], one16, mask=tail_mask)


def _scatter_wait(acc_sh, ibd, rowss, ssem, pset, b):
    pltpu.make_async_copy(
        rowss[b], acc_sh.at[ibd.at[pl.ds(pset * SK + b * K, K)]],
        ssem.at[b]).wait()


def _gathers(x_ref, acc_sh, ibs, ibd, rowss, gsem, ssem, pset, wait_pset):
    dg = []
    for b in range(S):
        if wait_pset is not None:
            _scatter_wait(acc_sh, ibd, rowss, ssem, wait_pset, b)
        dg.append(pltpu.async_copy(
            x_ref.at[ibs.at[pl.ds(pset * SK + b * K, K)]],
            rowss[b], gsem.at[b]))
    return dg


def _scatters(acc_sh, ibd, rowss, ssem, pset, dg):
    for b in range(S):
        dg[b].wait()
        pltpu.async_copy(rowss[b],
                         acc_sh.at[ibd.at[pl.ds(pset * SK + b * K, K)]],
                         ssem.at[b], add=True)


def _edge_loop(x_ref, esrc_ref, edst_ref, acc_sh, hist, ibs, ibd, rowss,
               isem, gsem, ssem, e_base, n_sg):
    one16 = jnp.full((16,), 1.0, jnp.float32)
    n_pairs = n_sg // 2

    _idx_issue(esrc_ref, edst_ref, ibs, ibd, isem, 0, e_base)

    _idx_wait(esrc_ref, edst_ref, ibs, ibd, isem, 0, e_base)
    dg = _gathers(x_ref, acc_sh, ibs, ibd, rowss, gsem, ssem, 0, None)
    _idx_issue(esrc_ref, edst_ref, ibs, ibd, isem, 1, e_base + SK)
    if hist is not None:
        _hist_update(hist, ibd, 0, one16)
    _scatters(acc_sh, ibd, rowss, ssem, 0, dg)

    _idx_wait(esrc_ref, edst_ref, ibs, ibd, isem, 1, e_base + SK)
    dg = _gathers(x_ref, acc_sh, ibs, ibd, rowss, gsem, ssem, 1, 0)
    _idx_issue(esrc_ref, edst_ref, ibs, ibd, isem, 0, e_base + 2 * SK)
    if hist is not None:
        _hist_update(hist, ibd, 1, one16)
    _scatters(acc_sh, ibd, rowss, ssem, 1, dg)

    def pair(tt, c):
        off0 = e_base + (2 * tt) * SK
        _idx_wait(esrc_ref, edst_ref, ibs, ibd, isem, 0, off0)
        d = _gathers(x_ref, acc_sh, ibs, ibd, rowss, gsem, ssem, 0, 1)
        _idx_issue(esrc_ref, edst_ref, ibs, ibd, isem, 1, off0 + SK)
        if hist is not None:
            _hist_update(hist, ibd, 0, one16)
        _scatters(acc_sh, ibd, rowss, ssem, 0, d)

        _idx_wait(esrc_ref, edst_ref, ibs, ibd, isem, 1, off0 + SK)
        d = _gathers(x_ref, acc_sh, ibs, ibd, rowss, gsem, ssem, 1, 0)

        @pl.when(tt + 1 < n_pairs)
        def _():
            _idx_issue(esrc_ref, edst_ref, ibs, ibd, isem, 0, off0 + 2 * SK)
        if hist is not None:
            _hist_update(hist, ibd, 1, one16)
        _scatters(acc_sh, ibd, rowss, ssem, 1, d)
        return c
    lax.fori_loop(1, n_pairs, pair, 0)

    for b in range(S):
        _scatter_wait(acc_sh, ibd, rowss, ssem, 1, b)


def _merge_hist_to_inv(hist, parts_ref, macc, mtmp, zb, inv_ref, c, s):
    cbase = c * NS * N_NODE
    pltpu.sync_copy(hist, parts_ref.at[pl.ds(cbase + s * N_NODE, N_NODE)])
    plsc.subcore_barrier()

    def stripe_fn(r0, n):
        nv = n // 16
        pltpu.sync_copy(parts_ref.at[pl.ds(cbase + r0, n)],
                        macc.at[pl.ds(0, n)])
        for t2 in range(1, NS):
            pltpu.sync_copy(parts_ref.at[pl.ds(cbase + t2 * N_NODE + r0, n)],
                            mtmp.at[pl.ds(0, n)])

            def addv(i, cc):
                macc[pl.ds(i * 16, 16)] = (macc[pl.ds(i * 16, 16)]
                                           + mtmp[pl.ds(i * 16, 16)])
                return cc
            lax.fori_loop(0, nv, addv, 0)

        def invv(i, cc):
            v = macc[pl.ds(i * 16, 16)]
            macc[pl.ds(i * 16, 16)] = 1.0 / jnp.maximum(v, 1.0)
            return cc
        lax.fori_loop(0, nv, invv, 0)

        n_bc = max(n // BC, 1)
        bc = min(BC, n)
        for jc in range(n_bc):
            base = jc * bc

            def bgroup(q, cc):
                vec = macc[pl.ds(base + q * 16, 16)]
                for l in range(16):
                    v16 = jnp.full((16,), vec[l], jnp.float32)
                    for j in range(D // 16):
                        zb[q * 16 + l, pl.ds(j * 16, 16)] = v16
                return cc
            lax.fori_loop(0, bc // 16, bgroup, 0)
            pltpu.sync_copy(zb.at[pl.ds(0, bc)],
                            inv_ref.at[c, pl.ds(r0 + base, bc)])
    _stripe(stripe_fn, s)


def _sc_layer0_body(xa_ref, xp_ref, eas_ref, ead_ref, eps_ref, epd_ref,
                    agg_ref, inv_ref, parts_ref,
                    acc_sh, hist, ibs, ibd, rowss, macc, mtmp,
                    zb, isem, gsem, ssem):
    c = lax.axis_index("c")
    s = lax.axis_index("s")
    _zero_acc(zb, acc_sh, s)

    def zhist(i, cc):
        hist[pl.ds(i * 16, 16)] = jnp.zeros((16,), jnp.float32)
        return cc
    lax.fori_loop(0, N_NODE // 16, zhist, 0)
    plsc.subcore_barrier()

    e_per_tile = NE // NS
    n_sg = e_per_tile // SK
    e_base = s * e_per_tile

    @pl.when(c == 0)
    def _():
        _edge_loop(xa_ref, eas_ref, ead_ref, acc_sh, hist, ibs, ibd,
                   rowss, isem, gsem, ssem, e_base, n_sg)

    @pl.when(c == 1)
    def _():
        _edge_loop(xp_ref, eps_ref, epd_ref, acc_sh, hist, ibs, ibd,
                   rowss, isem, gsem, ssem, e_base, n_sg)

    plsc.subcore_barrier()

    def out(r0, n):
        pltpu.sync_copy(acc_sh.at[pl.ds(r0, n)], agg_ref.at[c, pl.ds(r0, n)])
    _stripe(out, s)

    _merge_hist_to_inv(hist, parts_ref, macc, mtmp, zb, inv_ref, c, s)


def _sc_layer1_body(xp_ref, eps_ref, epd_ref, part_ref,
                    acc_sh, ibs, ibd, rowss, zb, isem, gsem, ssem):
    c = lax.axis_index("c")
    s = lax.axis_index("s")
    _zero_acc(zb, acc_sh, s)
    plsc.subcore_barrier()

    e_per_core = NE // 2
    e_per_tile = e_per_core // NS
    n_sg = e_per_tile // SK
    e_base = c * e_per_core + s * e_per_tile
    _edge_loop(xp_ref, eps_ref, epd_ref, acc_sh, None, ibs, ibd, rowss,
               isem, gsem, ssem, e_base, n_sg)

    plsc.subcore_barrier()

    def out(r0, n):
        pltpu.sync_copy(acc_sh.at[pl.ds(r0, n)], part_ref.at[c, pl.ds(r0, n)])
    _stripe(out, s)


_sc_layer0 = pl.kernel(
    _sc_layer0_body,
    out_type=[
        jax.ShapeDtypeStruct((2, N_NODE, D), jnp.float32),
        jax.ShapeDtypeStruct((2, N_NODE, D), jnp.float32),
        jax.ShapeDtypeStruct((2 * NS * N_NODE,), jnp.float32),
    ],
    mesh=plsc.VectorSubcoreMesh(core_axis_name="c", subcore_axis_name="s",
                                num_cores=2, num_subcores=NS),
    scratch_types=[
        pltpu.VMEM_SHARED((N_NODE, D), jnp.float32),
        pltpu.VMEM((N_NODE,), jnp.float32),
        pltpu.VMEM((2 * SK,), jnp.int32),
        pltpu.VMEM((2 * SK,), jnp.int32),
        tuple(pltpu.VMEM((K, D), jnp.float32) for _ in range(S)),
        pltpu.VMEM((ROWS_PER_TILE,), jnp.float32),
        pltpu.VMEM((ROWS_PER_TILE,), jnp.float32),
        pltpu.VMEM((BC, D), jnp.float32),
        pltpu.SemaphoreType.DMA((2,)),
        pltpu.SemaphoreType.DMA((S,)),
        pltpu.SemaphoreType.DMA((S,)),
    ],
    compiler_params=pltpu.CompilerParams(needs_layout_passes=False),
)

_sc_layer1 = pl.kernel(
    _sc_layer1_body,
    out_type=[
        jax.ShapeDtypeStruct((2, N_NODE, D), jnp.float32),
    ],
    mesh=plsc.VectorSubcoreMesh(core_axis_name="c", subcore_axis_name="s",
                                num_cores=2, num_subcores=NS),
    scratch_types=[
        pltpu.VMEM_SHARED((N_NODE, D), jnp.float32),
        pltpu.VMEM((2 * SK,), jnp.int32),
        pltpu.VMEM((2 * SK,), jnp.int32),
        tuple(pltpu.VMEM((K, D), jnp.float32) for _ in range(S)),
        pltpu.VMEM((BC, D), jnp.float32),
        pltpu.SemaphoreType.DMA((2,)),
        pltpu.SemaphoreType.DMA((S,)),
        pltpu.SemaphoreType.DMA((S,)),
    ],
    compiler_params=pltpu.CompilerParams(needs_layout_passes=False),
)


def _leaky(x):
    return jnp.where(x >= 0, x, 0.01 * x)


def _tc_sage_kern(agg_ref, inv_ref, x_ref, wm_ref, wr_ref, b_ref, out_ref):
    mean = agg_ref[...] * inv_ref[...]
    h = (jnp.dot(mean, wm_ref[...], preferred_element_type=jnp.float32)
         + b_ref[...]
         + jnp.dot(x_ref[...], wr_ref[...], preferred_element_type=jnp.float32))
    out_ref[...] = _leaky(h)


def _tc_layer1_kern(part_ref, inv_ref, xa_ref, wm_ref, wr_ref, b_ref,
                    wl_ref, bl_ref, out_ref):
    mean = (part_ref[0] + part_ref[1]) * inv_ref[0]
    h = (jnp.dot(mean, wm_ref[...], preferred_element_type=jnp.float32)
         + b_ref[...]
         + jnp.dot(xa_ref[...], wr_ref[...], preferred_element_type=jnp.float32))
    h = _leaky(h)
    out_ref[...] = (jnp.dot(h, wl_ref[...], preferred_element_type=jnp.float32)
                    + bl_ref[...])


_TC_R = 1000


def _tc_sage(agg, inv, x, wm, wr, b):
    grid = (N_NODE // _TC_R,)
    return pl.pallas_call(
        _tc_sage_kern,
        grid=grid,
        in_specs=[
            pl.BlockSpec((_TC_R, D), lambda r: (r, 0)),
            pl.BlockSpec((_TC_R, D), lambda r: (r, 0)),
            pl.BlockSpec((_TC_R, D), lambda r: (r, 0)),
            pl.BlockSpec((D, D), lambda r: (0, 0)),
            pl.BlockSpec((D, D), lambda r: (0, 0)),
            pl.BlockSpec((1, D), lambda r: (0, 0)),
        ],
        out_specs=pl.BlockSpec((_TC_R, D), lambda r: (r, 0)),
        out_shape=jax.ShapeDtypeStruct((N_NODE, D), jnp.float32),
    )(agg, inv, x, wm, wr, b)


def _tc_layer1(part, inv, xa, wm, wr, b, wl, bl):
    grid = (N_NODE // _TC_R,)
    return pl.pallas_call(
        _tc_layer1_kern,
        grid=grid,
        in_specs=[
            pl.BlockSpec((2, _TC_R, D), lambda r: (0, r, 0)),
            pl.BlockSpec((1, _TC_R, D), lambda r: (1, r, 0)),
            pl.BlockSpec((_TC_R, D), lambda r: (r, 0)),
            pl.BlockSpec((D, D), lambda r: (0, 0)),
            pl.BlockSpec((D, D), lambda r: (0, 0)),
            pl.BlockSpec((1, D), lambda r: (0, 0)),
            pl.BlockSpec((D, D_OUT), lambda r: (0, 0)),
            pl.BlockSpec((1, D_OUT), lambda r: (0, 0)),
        ],
        out_specs=pl.BlockSpec((_TC_R, D_OUT), lambda r: (r, 0)),
        out_shape=jax.ShapeDtypeStruct((N_NODE, D_OUT), jnp.float32),
    )(part, inv, xa, wm, wr, b, wl, bl)


def kernel(x_author, x_paper, edge_index_a2p, edge_index_p2a,
           W_msg_l0_a2p, b_l0_a2p, W_root_l0_a2p,
           W_msg_l0_p2a, b_l0_p2a, W_root_l0_p2a,
           W_msg_l1_a2p, b_l1_a2p, W_root_l1_a2p,
           W_msg_l1_p2a, b_l1_p2a, W_root_l1_p2a,
           W_lin, b_lin):
    ea_src, ea_dst = edge_index_a2p[0], edge_index_a2p[1]
    ep_src, ep_dst = edge_index_p2a[0], edge_index_p2a[1]
    agg0, inv, _ = _sc_layer0(x_author, x_paper, ea_src, ea_dst, ep_src,
                              ep_dst)

    new_p = _tc_sage(agg0[0], inv[0], x_paper, W_msg_l0_a2p, W_root_l0_a2p,
                     b_l0_a2p.reshape(1, D))
    part1 = _sc_layer1(new_p, ep_src, ep_dst)[0]
    new_a = _tc_sage(agg0[1], inv[1], x_author, W_msg_l0_p2a, W_root_l0_p2a,
                     b_l0_p2a.reshape(1, D))
    return _tc_layer1(part1, inv, new_a,
                      W_msg_l1_p2a, W_root_l1_p2a, b_l1_p2a.reshape(1, D),
                      W_lin, b_lin.reshape(1, D_OUT))


# speedup vs baseline: 1.0571x; 1.0571x over previous
import jax
import jax.numpy as jnp
from jax import lax
from jax.experimental import pallas as pl
from jax.experimental.pallas import tpu as pltpu
from jax.experimental.pallas import tpu_sc as plsc

N_NODE = 10000
NE = 320000
D = 128
D_OUT = 64
NS = 16
K = 40
ROWS_PER_TILE = 624
TAIL_ROWS = N_NODE - NS * ROWS_PER_TILE
TAIL_BASE = NS * ROWS_PER_TILE
BC = 16


def _stripe(fn, s):
    fn(s * ROWS_PER_TILE, ROWS_PER_TILE)

    @pl.when(s == NS - 1)
    def _():
        fn(TAIL_BASE, TAIL_ROWS)


def _zero_acc(zb, acc_sh, s):
    zv = jnp.zeros((16,), jnp.float32)

    def zrow(i, c):
        for j in range(D // 16):
            zb[i, pl.ds(j * 16, 16)] = zv
        return c
    lax.fori_loop(0, BC, zrow, 0)

    def z(r0, n):
        if n == ROWS_PER_TILE:
            for j in range(ROWS_PER_TILE // BC):
                pltpu.sync_copy(zb, acc_sh.at[pl.ds(r0 + j * BC, BC)])
        else:
            pltpu.sync_copy(zb.at[pl.ds(0, n)], acc_sh.at[pl.ds(r0, n)])
    _stripe(z, s)


S = 5
SK = K * S


def _idx_issue(esrc_ref, edst_ref, ibs, ibd, isem, pset, off):
    pltpu.async_copy(esrc_ref.at[pl.ds(off, SK)],
                     ibs.at[pl.ds(pset * SK, SK)], isem.at[pset])
    pltpu.async_copy(edst_ref.at[pl.ds(off, SK)],
                     ibd.at[pl.ds(pset * SK, SK)], isem.at[pset])


def _idx_wait(esrc_ref, edst_ref, ibs, ibd, isem, pset, off):
    pltpu.make_async_copy(esrc_ref.at[pl.ds(off, SK)],
                          ibs.at[pl.ds(pset * SK, SK)], isem.at[pset]).wait()
    pltpu.make_async_copy(edst_ref.at[pl.ds(off, SK)],
                          ibd.at[pl.ds(pset * SK, SK)], isem.at[pset]).wait()


def _hist_update(hist, ibd, pset, one16):
    for b in range(S):
        base = pset * SK + b * K
        for j in range(K // 16):
            iv = ibd[pl.ds(base + j * 16, 16)]
            plsc.addupdate_scatter(hist, [iv], one16)
        if K % 16:
            iv = ibd[pl.ds(base + K - 16, 16)]
            tail_mask = lax.iota(jnp.int32, 16) >= (16 - K % 16)
            plsc.addupdate_scatter(hist, [iv], one16, mask=tail_mask)


def _scatter_wait(acc_sh, ibd, rowss, ssem, pset, b):
    pltpu.make_async_copy(
        rowss[b], acc_sh.at[ibd.at[pl.ds(pset * SK + b * K, K)]],
        ssem.at[b]).wait()


def _gathers(x_ref, acc_sh, ibs, ibd, rowss, gsem, ssem, pset, wait_pset):
    dg = []
    for b in range(S):
        if wait_pset is not None:
            _scatter_wait(acc_sh, ibd, rowss, ssem, wait_pset, b)
        dg.append(pltpu.async_copy(
            x_ref.at[ibs.at[pl.ds(pset * SK + b * K, K)]],
            rowss[b], gsem.at[b]))
    return dg


def _scatters(acc_sh, ibd, rowss, ssem, pset, dg):
    for b in range(S):
        dg[b].wait()
        pltpu.async_copy(rowss[b],
                         acc_sh.at[ibd.at[pl.ds(pset * SK + b * K, K)]],
                         ssem.at[b], add=True)


def _edge_loop(x_ref, esrc_ref, edst_ref, acc_sh, hist, ibs, ibd, rowss,
               isem, gsem, ssem, e_base, n_sg):
    one16 = jnp.full((16,), 1.0, jnp.float32)
    n_pairs = n_sg // 2

    _idx_issue(esrc_ref, edst_ref, ibs, ibd, isem, 0, e_base)

    _idx_wait(esrc_ref, edst_ref, ibs, ibd, isem, 0, e_base)
    dg = _gathers(x_ref, acc_sh, ibs, ibd, rowss, gsem, ssem, 0, None)
    _idx_issue(esrc_ref, edst_ref, ibs, ibd, isem, 1, e_base + SK)
    if hist is not None:
        _hist_update(hist, ibd, 0, one16)
    _scatters(acc_sh, ibd, rowss, ssem, 0, dg)

    _idx_wait(esrc_ref, edst_ref, ibs, ibd, isem, 1, e_base + SK)
    dg = _gathers(x_ref, acc_sh, ibs, ibd, rowss, gsem, ssem, 1, 0)
    _idx_issue(esrc_ref, edst_ref, ibs, ibd, isem, 0, e_base + 2 * SK)
    if hist is not None:
        _hist_update(hist, ibd, 1, one16)
    _scatters(acc_sh, ibd, rowss, ssem, 1, dg)

    def pair(tt, c):
        off0 = e_base + (2 * tt) * SK
        _idx_wait(esrc_ref, edst_ref, ibs, ibd, isem, 0, off0)
        d = _gathers(x_ref, acc_sh, ibs, ibd, rowss, gsem, ssem, 0, 1)
        _idx_issue(esrc_ref, edst_ref, ibs, ibd, isem, 1, off0 + SK)
        if hist is not None:
            _hist_update(hist, ibd, 0, one16)
        _scatters(acc_sh, ibd, rowss, ssem, 0, d)

        _idx_wait(esrc_ref, edst_ref, ibs, ibd, isem, 1, off0 + SK)
        d = _gathers(x_ref, acc_sh, ibs, ibd, rowss, gsem, ssem, 1, 0)

        @pl.when(2 * tt + 2 < n_sg)
        def _():
            _idx_issue(esrc_ref, edst_ref, ibs, ibd, isem, 0, off0 + 2 * SK)
        if hist is not None:
            _hist_update(hist, ibd, 1, one16)
        _scatters(acc_sh, ibd, rowss, ssem, 1, d)
        return c
    lax.fori_loop(1, n_pairs, pair, 0)

    if n_sg % 2:
        off_last = e_base + (n_sg - 1) * SK
        _idx_wait(esrc_ref, edst_ref, ibs, ibd, isem, 0, off_last)
        dg = _gathers(x_ref, acc_sh, ibs, ibd, rowss, gsem, ssem, 0, 1)
        if hist is not None:
            _hist_update(hist, ibd, 0, one16)
        _scatters(acc_sh, ibd, rowss, ssem, 0, dg)

    for b in range(S):
        _scatter_wait(acc_sh, ibd, rowss, ssem, n_sg % 2 ^ 1, b)


def _merge_hist_to_inv(hist, parts_ref, macc, mtmp, zb, inv_ref, c, s):
    cbase = c * NS * N_NODE
    pltpu.sync_copy(hist, parts_ref.at[pl.ds(cbase + s * N_NODE, N_NODE)])
    plsc.subcore_barrier()

    def stripe_fn(r0, n):
        nv = n // 16
        pltpu.sync_copy(parts_ref.at[pl.ds(cbase + r0, n)],
                        macc.at[pl.ds(0, n)])
        for t2 in range(1, NS):
            pltpu.sync_copy(parts_ref.at[pl.ds(cbase + t2 * N_NODE + r0, n)],
                            mtmp.at[pl.ds(0, n)])

            def addv(i, cc):
                macc[pl.ds(i * 16, 16)] = (macc[pl.ds(i * 16, 16)]
                                           + mtmp[pl.ds(i * 16, 16)])
                return cc
            lax.fori_loop(0, nv, addv, 0)

        def invv(i, cc):
            v = macc[pl.ds(i * 16, 16)]
            macc[pl.ds(i * 16, 16)] = 1.0 / jnp.maximum(v, 1.0)
            return cc
        lax.fori_loop(0, nv, invv, 0)

        n_bc = max(n // BC, 1)
        bc = min(BC, n)
        for jc in range(n_bc):
            base = jc * bc

            def bgroup(q, cc):
                vec = macc[pl.ds(base + q * 16, 16)]
                for l in range(16):
                    v16 = jnp.full((16,), vec[l], jnp.float32)
                    for j in range(D // 16):
                        zb[q * 16 + l, pl.ds(j * 16, 16)] = v16
                return cc
            lax.fori_loop(0, bc // 16, bgroup, 0)
            pltpu.sync_copy(zb.at[pl.ds(0, bc)],
                            inv_ref.at[c, pl.ds(r0 + base, bc)])
    _stripe(stripe_fn, s)


def _sc_layer0_body(xa_ref, xp_ref, eas_ref, ead_ref, eps_ref, epd_ref,
                    agg_ref, inv_ref, parts_ref,
                    acc_sh, hist, ibs, ibd, rowss, macc, mtmp,
                    zb, isem, gsem, ssem):
    c = lax.axis_index("c")
    s = lax.axis_index("s")
    _zero_acc(zb, acc_sh, s)

    def zhist(i, cc):
        hist[pl.ds(i * 16, 16)] = jnp.zeros((16,), jnp.float32)
        return cc
    lax.fori_loop(0, N_NODE // 16, zhist, 0)
    plsc.subcore_barrier()

    e_per_tile = NE // NS
    n_sg = e_per_tile // SK
    e_base = s * e_per_tile

    @pl.when(c == 0)
    def _():
        _edge_loop(xa_ref, eas_ref, ead_ref, acc_sh, hist, ibs, ibd,
                   rowss, isem, gsem, ssem, e_base, n_sg)

    @pl.when(c == 1)
    def _():
        _edge_loop(xp_ref, eps_ref, epd_ref, acc_sh, hist, ibs, ibd,
                   rowss, isem, gsem, ssem, e_base, n_sg)

    plsc.subcore_barrier()

    def out(r0, n):
        pltpu.sync_copy(acc_sh.at[pl.ds(r0, n)], agg_ref.at[c, pl.ds(r0, n)])
    _stripe(out, s)

    _merge_hist_to_inv(hist, parts_ref, macc, mtmp, zb, inv_ref, c, s)


def _sc_layer1_body(xp_ref, eps_ref, epd_ref, part_ref,
                    acc_sh, ibs, ibd, rowss, zb, isem, gsem, ssem):
    c = lax.axis_index("c")
    s = lax.axis_index("s")
    _zero_acc(zb, acc_sh, s)
    plsc.subcore_barrier()

    e_per_core = NE // 2
    e_per_tile = e_per_core // NS
    n_sg = e_per_tile // SK
    e_base = c * e_per_core + s * e_per_tile
    _edge_loop(xp_ref, eps_ref, epd_ref, acc_sh, None, ibs, ibd, rowss,
               isem, gsem, ssem, e_base, n_sg)

    plsc.subcore_barrier()

    def out(r0, n):
        pltpu.sync_copy(acc_sh.at[pl.ds(r0, n)], part_ref.at[c, pl.ds(r0, n)])
    _stripe(out, s)


_sc_layer0 = pl.kernel(
    _sc_layer0_body,
    out_type=[
        jax.ShapeDtypeStruct((2, N_NODE, D), jnp.float32),
        jax.ShapeDtypeStruct((2, N_NODE, D), jnp.float32),
        jax.ShapeDtypeStruct((2 * NS * N_NODE,), jnp.float32),
    ],
    mesh=plsc.VectorSubcoreMesh(core_axis_name="c", subcore_axis_name="s",
                                num_cores=2, num_subcores=NS),
    scratch_types=[
        pltpu.VMEM_SHARED((N_NODE, D), jnp.float32),
        pltpu.VMEM((N_NODE,), jnp.float32),
        pltpu.VMEM((2 * SK,), jnp.int32),
        pltpu.VMEM((2 * SK,), jnp.int32),
        tuple(pltpu.VMEM((K, D), jnp.float32) for _ in range(S)),
        pltpu.VMEM((ROWS_PER_TILE,), jnp.float32),
        pltpu.VMEM((ROWS_PER_TILE,), jnp.float32),
        pltpu.VMEM((BC, D), jnp.float32),
        pltpu.SemaphoreType.DMA((2,)),
        pltpu.SemaphoreType.DMA((S,)),
        pltpu.SemaphoreType.DMA((S,)),
    ],
    compiler_params=pltpu.CompilerParams(needs_layout_passes=False),
)

_sc_layer1 = pl.kernel(
    _sc_layer1_body,
    out_type=[
        jax.ShapeDtypeStruct((2, N_NODE, D), jnp.float32),
    ],
    mesh=plsc.VectorSubcoreMesh(core_axis_name="c", subcore_axis_name="s",
                                num_cores=2, num_subcores=NS),
    scratch_types=[
        pltpu.VMEM_SHARED((N_NODE, D), jnp.float32),
        pltpu.VMEM((2 * SK,), jnp.int32),
        pltpu.VMEM((2 * SK,), jnp.int32),
        tuple(pltpu.VMEM((K, D), jnp.float32) for _ in range(S)),
        pltpu.VMEM((BC, D), jnp.float32),
        pltpu.SemaphoreType.DMA((2,)),
        pltpu.SemaphoreType.DMA((S,)),
        pltpu.SemaphoreType.DMA((S,)),
    ],
    compiler_params=pltpu.CompilerParams(needs_layout_passes=False),
)


def _leaky(x):
    return jnp.where(x >= 0, x, 0.01 * x)


def _tc_layer0_kern(agg_ref, inv_ref, xp_ref, xa_ref, wm_ref, wr_ref, b_ref,
                    newp_ref, newa_ref):
    mean_p = agg_ref[0] * inv_ref[0]
    h = (jnp.dot(mean_p, wm_ref[0], preferred_element_type=jnp.float32)
         + b_ref[0:1, :]
         + jnp.dot(xp_ref[...], wr_ref[0], preferred_element_type=jnp.float32))
    newp_ref[...] = _leaky(h)
    mean_a = agg_ref[1] * inv_ref[1]
    h = (jnp.dot(mean_a, wm_ref[1], preferred_element_type=jnp.float32)
         + b_ref[1:2, :]
         + jnp.dot(xa_ref[...], wr_ref[1], preferred_element_type=jnp.float32))
    newa_ref[...] = _leaky(h)


def _tc_layer1_kern(part_ref, inv_ref, xa_ref, wm_ref, wr_ref, b_ref,
                    wl_ref, bl_ref, out_ref):
    mean = (part_ref[0] + part_ref[1]) * inv_ref[0]
    h = (jnp.dot(mean, wm_ref[...], preferred_element_type=jnp.float32)
         + b_ref[...]
         + jnp.dot(xa_ref[...], wr_ref[...], preferred_element_type=jnp.float32))
    h = _leaky(h)
    out_ref[...] = (jnp.dot(h, wl_ref[...], preferred_element_type=jnp.float32)
                    + bl_ref[...])


_TC_R = 2000


def _tc_layer0(agg, inv, x_paper, x_author, wm, wr, b):
    grid = (N_NODE // _TC_R,)
    return pl.pallas_call(
        _tc_layer0_kern,
        grid=grid,
        in_specs=[
            pl.BlockSpec((2, _TC_R, D), lambda r: (0, r, 0)),
            pl.BlockSpec((2, _TC_R, D), lambda r: (0, r, 0)),
            pl.BlockSpec((_TC_R, D), lambda r: (r, 0)),
            pl.BlockSpec((_TC_R, D), lambda r: (r, 0)),
            pl.BlockSpec((2, D, D), lambda r: (0, 0, 0)),
            pl.BlockSpec((2, D, D), lambda r: (0, 0, 0)),
            pl.BlockSpec((2, D), lambda r: (0, 0)),
        ],
        out_specs=[
            pl.BlockSpec((_TC_R, D), lambda r: (r, 0)),
            pl.BlockSpec((_TC_R, D), lambda r: (r, 0)),
        ],
        out_shape=[
            jax.ShapeDtypeStruct((N_NODE, D), jnp.float32),
            jax.ShapeDtypeStruct((N_NODE, D), jnp.float32),
        ],
    )(agg, inv, x_paper, x_author, wm, wr, b)


def _tc_layer1(part, inv, xa, wm, wr, b, wl, bl):
    grid = (N_NODE // _TC_R,)
    return pl.pallas_call(
        _tc_layer1_kern,
        grid=grid,
        in_specs=[
            pl.BlockSpec((2, _TC_R, D), lambda r: (0, r, 0)),
            pl.BlockSpec((1, _TC_R, D), lambda r: (1, r, 0)),
            pl.BlockSpec((_TC_R, D), lambda r: (r, 0)),
            pl.BlockSpec((D, D), lambda r: (0, 0)),
            pl.BlockSpec((D, D), lambda r: (0, 0)),
            pl.BlockSpec((1, D), lambda r: (0, 0)),
            pl.BlockSpec((D, D_OUT), lambda r: (0, 0)),
            pl.BlockSpec((1, D_OUT), lambda r: (0, 0)),
        ],
        out_specs=pl.BlockSpec((_TC_R, D_OUT), lambda r: (r, 0)),
        out_shape=jax.ShapeDtypeStruct((N_NODE, D_OUT), jnp.float32),
    )(part, inv, xa, wm, wr, b, wl, bl)


def kernel(x_author, x_paper, edge_index_a2p, edge_index_p2a,
           W_msg_l0_a2p, b_l0_a2p, W_root_l0_a2p,
           W_msg_l0_p2a, b_l0_p2a, W_root_l0_p2a,
           W_msg_l1_a2p, b_l1_a2p, W_root_l1_a2p,
           W_msg_l1_p2a, b_l1_p2a, W_root_l1_p2a,
           W_lin, b_lin):
    ea_src, ea_dst = edge_index_a2p[0], edge_index_a2p[1]
    ep_src, ep_dst = edge_index_p2a[0], edge_index_p2a[1]
    agg0, inv, _ = _sc_layer0(x_author, x_paper, ea_src, ea_dst, ep_src,
                              ep_dst)

    wm0 = jnp.stack([W_msg_l0_a2p, W_msg_l0_p2a])
    wr0 = jnp.stack([W_root_l0_a2p, W_root_l0_p2a])
    b0 = jnp.stack([b_l0_a2p, b_l0_p2a])
    new_p, new_a = _tc_layer0(agg0, inv, x_paper, x_author, wm0, wr0, b0)

    part1 = _sc_layer1(new_p, ep_src, ep_dst)[0]
    return _tc_layer1(part1, inv, new_a,
                      W_msg_l1_p2a, W_root_l1_p2a, b_l1_p2a.reshape(1, D),
                      W_lin, b_lin.reshape(1, D_OUT))


# speedup vs baseline: 1.0662x; 1.0086x over previous
import jax
import jax.numpy as jnp
from jax import lax
from jax.experimental import pallas as pl
from jax.experimental.pallas import tpu as pltpu
from jax.experimental.pallas import tpu_sc as plsc

N_NODE = 10000
NE = 320000
D = 128
D_OUT = 64
NS = 16
K = 40
ROWS_PER_TILE = 624
TAIL_ROWS = N_NODE - NS * ROWS_PER_TILE
TAIL_BASE = NS * ROWS_PER_TILE
BC = 16


def _stripe(fn, s):
    fn(s * ROWS_PER_TILE, ROWS_PER_TILE)

    @pl.when(s == NS - 1)
    def _():
        fn(TAIL_BASE, TAIL_ROWS)


def _zero_acc(zb, acc_sh, s, zsem):
    zv = jnp.zeros((16,), jnp.float32)

    def zrow(i, c):
        for j in range(D // 16):
            zb[i, pl.ds(j * 16, 16)] = zv
        return c
    lax.fori_loop(0, BC, zrow, 0)

    def z(r0, n):
        if n == ROWS_PER_TILE:
            ds = [pltpu.async_copy(zb, acc_sh.at[pl.ds(r0 + j * BC, BC)],
                                   zsem)
                  for j in range(ROWS_PER_TILE // BC)]
            for d in ds:
                d.wait()
        else:
            pltpu.sync_copy(zb.at[pl.ds(0, n)], acc_sh.at[pl.ds(r0, n)])
    _stripe(z, s)


S = 5
SK = K * S


def _idx_issue(esrc_ref, edst_ref, ibs, ibd, isem, pset, off):
    pltpu.async_copy(esrc_ref.at[pl.ds(off, SK)],
                     ibs.at[pl.ds(pset * SK, SK)], isem.at[pset])
    pltpu.async_copy(edst_ref.at[pl.ds(off, SK)],
                     ibd.at[pl.ds(pset * SK, SK)], isem.at[pset])


def _idx_wait(esrc_ref, edst_ref, ibs, ibd, isem, pset, off):
    pltpu.make_async_copy(esrc_ref.at[pl.ds(off, SK)],
                          ibs.at[pl.ds(pset * SK, SK)], isem.at[pset]).wait()
    pltpu.make_async_copy(edst_ref.at[pl.ds(off, SK)],
                          ibd.at[pl.ds(pset * SK, SK)], isem.at[pset]).wait()


def _hist_update(hist, ibd, pset, one16):
    for b in range(S):
        base = pset * SK + b * K
        for j in range(K // 16):
            iv = ibd[pl.ds(base + j * 16, 16)]
            plsc.addupdate_scatter(hist, [iv], one16)
        if K % 16:
            iv = ibd[pl.ds(base + K - 16, 16)]
            tail_mask = lax.iota(jnp.int32, 16) >= (16 - K % 16)
            plsc.addupdate_scatter(hist, [iv], one16, mask=tail_mask)


def _scatter_wait(acc_sh, ibd, rowss, ssem, pset, b):
    pltpu.make_async_copy(
        rowss[b], acc_sh.at[ibd.at[pl.ds(pset * SK + b * K, K)]],
        ssem.at[b]).wait()


def _gathers(x_ref, acc_sh, ibs, ibd, rowss, gsem, ssem, pset, wait_pset):
    dg = []
    for b in range(S):
        if wait_pset is not None:
            _scatter_wait(acc_sh, ibd, rowss, ssem, wait_pset, b)
        dg.append(pltpu.async_copy(
            x_ref.at[ibs.at[pl.ds(pset * SK + b * K, K)]],
            rowss[b], gsem.at[b]))
    return dg


def _scatters(acc_sh, ibd, rowss, ssem, pset, dg):
    for b in range(S):
        dg[b].wait()
        pltpu.async_copy(rowss[b],
                         acc_sh.at[ibd.at[pl.ds(pset * SK + b * K, K)]],
                         ssem.at[b], add=True)


def _edge_loop(x_ref, esrc_ref, edst_ref, acc_sh, hist, ibs, ibd, rowss,
               isem, gsem, ssem, e_base, n_sg):
    one16 = jnp.full((16,), 1.0, jnp.float32)
    n_pairs = n_sg // 2

    _idx_issue(esrc_ref, edst_ref, ibs, ibd, isem, 0, e_base)

    _idx_wait(esrc_ref, edst_ref, ibs, ibd, isem, 0, e_base)
    dg = _gathers(x_ref, acc_sh, ibs, ibd, rowss, gsem, ssem, 0, None)
    _idx_issue(esrc_ref, edst_ref, ibs, ibd, isem, 1, e_base + SK)
    if hist is not None:
        _hist_update(hist, ibd, 0, one16)
    _scatters(acc_sh, ibd, rowss, ssem, 0, dg)

    _idx_wait(esrc_ref, edst_ref, ibs, ibd, isem, 1, e_base + SK)
    dg = _gathers(x_ref, acc_sh, ibs, ibd, rowss, gsem, ssem, 1, 0)
    _idx_issue(esrc_ref, edst_ref, ibs, ibd, isem, 0, e_base + 2 * SK)
    if hist is not None:
        _hist_update(hist, ibd, 1, one16)
    _scatters(acc_sh, ibd, rowss, ssem, 1, dg)

    def pair(tt, c):
        off0 = e_base + (2 * tt) * SK
        _idx_wait(esrc_ref, edst_ref, ibs, ibd, isem, 0, off0)
        d = _gathers(x_ref, acc_sh, ibs, ibd, rowss, gsem, ssem, 0, 1)
        _idx_issue(esrc_ref, edst_ref, ibs, ibd, isem, 1, off0 + SK)
        if hist is not None:
            _hist_update(hist, ibd, 0, one16)
        _scatters(acc_sh, ibd, rowss, ssem, 0, d)

        _idx_wait(esrc_ref, edst_ref, ibs, ibd, isem, 1, off0 + SK)
        d = _gathers(x_ref, acc_sh, ibs, ibd, rowss, gsem, ssem, 1, 0)

        @pl.when(2 * tt + 2 < n_sg)
        def _():
            _idx_issue(esrc_ref, edst_ref, ibs, ibd, isem, 0, off0 + 2 * SK)
        if hist is not None:
            _hist_update(hist, ibd, 1, one16)
        _scatters(acc_sh, ibd, rowss, ssem, 1, d)
        return c
    lax.fori_loop(1, n_pairs, pair, 0)

    if n_sg % 2:
        off_last = e_base + (n_sg - 1) * SK
        _idx_wait(esrc_ref, edst_ref, ibs, ibd, isem, 0, off_last)
        dg = _gathers(x_ref, acc_sh, ibs, ibd, rowss, gsem, ssem, 0, 1)
        if hist is not None:
            _hist_update(hist, ibd, 0, one16)
        _scatters(acc_sh, ibd, rowss, ssem, 0, dg)

    for b in range(S):
        _scatter_wait(acc_sh, ibd, rowss, ssem, n_sg % 2 ^ 1, b)


def _merge_hist_to_inv(hist, parts_ref, macc, mtmp, zb, inv_ref, c, s):
    cbase = c * NS * N_NODE
    pltpu.sync_copy(hist, parts_ref.at[pl.ds(cbase + s * N_NODE, N_NODE)])
    plsc.subcore_barrier()

    def stripe_fn(r0, n):
        nv = n // 16
        pltpu.sync_copy(parts_ref.at[pl.ds(cbase + r0, n)],
                        macc.at[pl.ds(0, n)])
        for t2 in range(1, NS):
            pltpu.sync_copy(parts_ref.at[pl.ds(cbase + t2 * N_NODE + r0, n)],
                            mtmp.at[pl.ds(0, n)])

            def addv(i, cc):
                macc[pl.ds(i * 16, 16)] = (macc[pl.ds(i * 16, 16)]
                                           + mtmp[pl.ds(i * 16, 16)])
                return cc
            lax.fori_loop(0, nv, addv, 0)

        def invv(i, cc):
            v = macc[pl.ds(i * 16, 16)]
            macc[pl.ds(i * 16, 16)] = 1.0 / jnp.maximum(v, 1.0)
            return cc
        lax.fori_loop(0, nv, invv, 0)

        n_bc = max(n // BC, 1)
        bc = min(BC, n)
        for jc in range(n_bc):
            base = jc * bc

            def bgroup(q, cc):
                vec = macc[pl.ds(base + q * 16, 16)]
                for l in range(16):
                    v16 = jnp.full((16,), vec[l], jnp.float32)
                    for j in range(D // 16):
                        zb[q * 16 + l, pl.ds(j * 16, 16)] = v16
                return cc
            lax.fori_loop(0, bc // 16, bgroup, 0)
            pltpu.sync_copy(zb.at[pl.ds(0, bc)],
                            inv_ref.at[c, pl.ds(r0 + base, bc)])
    _stripe(stripe_fn, s)


def _sc_layer0_body(xa_ref, xp_ref, eas_ref, ead_ref, eps_ref, epd_ref,
                    agg_ref, inv_ref, parts_ref,
                    acc_sh, hist, ibs, ibd, rowss, macc, mtmp,
                    zb, isem, gsem, ssem):
    c = lax.axis_index("c")
    s = lax.axis_index("s")
    _zero_acc(zb, acc_sh, s, isem.at[0])

    def zhist(i, cc):
        hist[pl.ds(i * 16, 16)] = jnp.zeros((16,), jnp.float32)
        return cc
    lax.fori_loop(0, N_NODE // 16, zhist, 0)
    plsc.subcore_barrier()

    e_per_tile = NE // NS
    n_sg = e_per_tile // SK
    e_base = s * e_per_tile

    @pl.when(c == 0)
    def _():
        _edge_loop(xa_ref, eas_ref, ead_ref, acc_sh, hist, ibs, ibd,
                   rowss, isem, gsem, ssem, e_base, n_sg)

    @pl.when(c == 1)
    def _():
        _edge_loop(xp_ref, eps_ref, epd_ref, acc_sh, hist, ibs, ibd,
                   rowss, isem, gsem, ssem, e_base, n_sg)

    plsc.subcore_barrier()

    def out(r0, n):
        pltpu.sync_copy(acc_sh.at[pl.ds(r0, n)], agg_ref.at[c, pl.ds(r0, n)])
    _stripe(out, s)

    _merge_hist_to_inv(hist, parts_ref, macc, mtmp, zb, inv_ref, c, s)


def _sc_layer1_body(xp_ref, eps_ref, epd_ref, part_ref,
                    acc_sh, ibs, ibd, rowss, zb, isem, gsem, ssem):
    c = lax.axis_index("c")
    s = lax.axis_index("s")
    _zero_acc(zb, acc_sh, s, isem.at[0])
    plsc.subcore_barrier()

    e_per_core = NE // 2
    e_per_tile = e_per_core // NS
    n_sg = e_per_tile // SK
    e_base = c * e_per_core + s * e_per_tile
    _edge_loop(xp_ref, eps_ref, epd_ref, acc_sh, None, ibs, ibd, rowss,
               isem, gsem, ssem, e_base, n_sg)

    plsc.subcore_barrier()

    def out(r0, n):
        pltpu.sync_copy(acc_sh.at[pl.ds(r0, n)], part_ref.at[c, pl.ds(r0, n)])
    _stripe(out, s)


_sc_layer0 = pl.kernel(
    _sc_layer0_body,
    out_type=[
        jax.ShapeDtypeStruct((2, N_NODE, D), jnp.float32),
        jax.ShapeDtypeStruct((2, N_NODE, D), jnp.float32),
        jax.ShapeDtypeStruct((2 * NS * N_NODE,), jnp.float32),
    ],
    mesh=plsc.VectorSubcoreMesh(core_axis_name="c", subcore_axis_name="s",
                                num_cores=2, num_subcores=NS),
    scratch_types=[
        pltpu.VMEM_SHARED((N_NODE, D), jnp.float32),
        pltpu.VMEM((N_NODE,), jnp.float32),
        pltpu.VMEM((2 * SK,), jnp.int32),
        pltpu.VMEM((2 * SK,), jnp.int32),
        tuple(pltpu.VMEM((K, D), jnp.float32) for _ in range(S)),
        pltpu.VMEM((ROWS_PER_TILE,), jnp.float32),
        pltpu.VMEM((ROWS_PER_TILE,), jnp.float32),
        pltpu.VMEM((BC, D), jnp.float32),
        pltpu.SemaphoreType.DMA((2,)),
        pltpu.SemaphoreType.DMA((S,)),
        pltpu.SemaphoreType.DMA((S,)),
    ],
    compiler_params=pltpu.CompilerParams(needs_layout_passes=False),
)

_sc_layer1 = pl.kernel(
    _sc_layer1_body,
    out_type=[
        jax.ShapeDtypeStruct((2, N_NODE, D), jnp.float32),
    ],
    mesh=plsc.VectorSubcoreMesh(core_axis_name="c", subcore_axis_name="s",
                                num_cores=2, num_subcores=NS),
    scratch_types=[
        pltpu.VMEM_SHARED((N_NODE, D), jnp.float32),
        pltpu.VMEM((2 * SK,), jnp.int32),
        pltpu.VMEM((2 * SK,), jnp.int32),
        tuple(pltpu.VMEM((K, D), jnp.float32) for _ in range(S)),
        pltpu.VMEM((BC, D), jnp.float32),
        pltpu.SemaphoreType.DMA((2,)),
        pltpu.SemaphoreType.DMA((S,)),
        pltpu.SemaphoreType.DMA((S,)),
    ],
    compiler_params=pltpu.CompilerParams(needs_layout_passes=False),
)


def _leaky(x):
    return jnp.where(x >= 0, x, 0.01 * x)


def _tc_layer0_kern(agg_ref, inv_ref, xp_ref, xa_ref, wm_ref, wr_ref, b_ref,
                    newp_ref, newa_ref):
    mean_p = agg_ref[0] * inv_ref[0]
    h = (jnp.dot(mean_p, wm_ref[0], preferred_element_type=jnp.float32)
         + b_ref[0:1, :]
         + jnp.dot(xp_ref[...], wr_ref[0], preferred_element_type=jnp.float32))
    newp_ref[...] = _leaky(h)
    mean_a = agg_ref[1] * inv_ref[1]
    h = (jnp.dot(mean_a, wm_ref[1], preferred_element_type=jnp.float32)
         + b_ref[1:2, :]
         + jnp.dot(xa_ref[...], wr_ref[1], preferred_element_type=jnp.float32))
    newa_ref[...] = _leaky(h)


def _tc_layer1_kern(part_ref, inv_ref, xa_ref, wm_ref, wr_ref, b_ref,
                    wl_ref, bl_ref, out_ref):
    mean = (part_ref[0] + part_ref[1]) * inv_ref[0]
    h = (jnp.dot(mean, wm_ref[...], preferred_element_type=jnp.float32)
         + b_ref[...]
         + jnp.dot(xa_ref[...], wr_ref[...], preferred_element_type=jnp.float32))
    h = _leaky(h)
    out_ref[...] = (jnp.dot(h, wl_ref[...], preferred_element_type=jnp.float32)
                    + bl_ref[...])


_TC_R = 2000


def _tc_layer0(agg, inv, x_paper, x_author, wm, wr, b):
    grid = (N_NODE // _TC_R,)
    return pl.pallas_call(
        _tc_layer0_kern,
        grid=grid,
        in_specs=[
            pl.BlockSpec((2, _TC_R, D), lambda r: (0, r, 0)),
            pl.BlockSpec((2, _TC_R, D), lambda r: (0, r, 0)),
            pl.BlockSpec((_TC_R, D), lambda r: (r, 0)),
            pl.BlockSpec((_TC_R, D), lambda r: (r, 0)),
            pl.BlockSpec((2, D, D), lambda r: (0, 0, 0)),
            pl.BlockSpec((2, D, D), lambda r: (0, 0, 0)),
            pl.BlockSpec((2, D), lambda r: (0, 0)),
        ],
        out_specs=[
            pl.BlockSpec((_TC_R, D), lambda r: (r, 0)),
            pl.BlockSpec((_TC_R, D), lambda r: (r, 0)),
        ],
        out_shape=[
            jax.ShapeDtypeStruct((N_NODE, D), jnp.float32),
            jax.ShapeDtypeStruct((N_NODE, D), jnp.float32),
        ],
    )(agg, inv, x_paper, x_author, wm, wr, b)


def _tc_layer1(part, inv, xa, wm, wr, b, wl, bl):
    grid = (N_NODE // _TC_R,)
    return pl.pallas_call(
        _tc_layer1_kern,
        grid=grid,
        in_specs=[
            pl.BlockSpec((2, _TC_R, D), lambda r: (0, r, 0)),
            pl.BlockSpec((1, _TC_R, D), lambda r: (1, r, 0)),
            pl.BlockSpec((_TC_R, D), lambda r: (r, 0)),
            pl.BlockSpec((D, D), lambda r: (0, 0)),
            pl.BlockSpec((D, D), lambda r: (0, 0)),
            pl.BlockSpec((1, D), lambda r: (0, 0)),
            pl.BlockSpec((D, D_OUT), lambda r: (0, 0)),
            pl.BlockSpec((1, D_OUT), lambda r: (0, 0)),
        ],
        out_specs=pl.BlockSpec((_TC_R, D_OUT), lambda r: (r, 0)),
        out_shape=jax.ShapeDtypeStruct((N_NODE, D_OUT), jnp.float32),
    )(part, inv, xa, wm, wr, b, wl, bl)


def kernel(x_author, x_paper, edge_index_a2p, edge_index_p2a,
           W_msg_l0_a2p, b_l0_a2p, W_root_l0_a2p,
           W_msg_l0_p2a, b_l0_p2a, W_root_l0_p2a,
           W_msg_l1_a2p, b_l1_a2p, W_root_l1_a2p,
           W_msg_l1_p2a, b_l1_p2a, W_root_l1_p2a,
           W_lin, b_lin):
    ea_src, ea_dst = edge_index_a2p[0], edge_index_a2p[1]
    ep_src, ep_dst = edge_index_p2a[0], edge_index_p2a[1]
    agg0, inv, _ = _sc_layer0(x_author, x_paper, ea_src, ea_dst, ep_src,
                              ep_dst)

    wm0 = jnp.stack([W_msg_l0_a2p, W_msg_l0_p2a])
    wr0 = jnp.stack([W_root_l0_a2p, W_root_l0_p2a])
    b0 = jnp.stack([b_l0_a2p, b_l0_p2a])
    new_p, new_a = _tc_layer0(agg0, inv, x_paper, x_author, wm0, wr0, b0)

    part1 = _sc_layer1(new_p, ep_src, ep_dst)[0]
    return _tc_layer1(part1, inv, new_a,
                      W_msg_l1_p2a, W_root_l1_p2a, b_l1_p2a.reshape(1, D),
                      W_lin, b_lin.reshape(1, D_OUT))
